# baseline (speedup 1.0000x reference)
import jax
import jax.numpy as jnp
from jax import lax
from jax.experimental import pallas as pl
from jax.experimental.pallas import tpu as pltpu
from jax.experimental.pallas import tpu_sc as plsc

_VOCAB = 100000
_NC = 2
_NS = 16
_NW = _NC * _NS
_VP = 100352
_SLICE = _VP // _NS

_ROWS = 16384
_PW = 50
_PS = 56
_CW = 200
_ROWS_PER_W = _ROWS // _NW
_PR = 128
_CR = 32
_PSTAGE = _PR * _PS
_PLOAD = _PR * _PW
_CSTAGE = _CR * _CW


def _sc_hist_body(preds_hbm, capt_hbm, pred_out, capt_out,
                  pred_acc, capt_acc,
                  pst0, pst1, cst0, cst1, ones_buf, zbuf,
                  sem0, sem1):
  c = lax.axis_index("c")
  s = lax.axis_index("s")
  w = c * _NS + s
  row0 = w * _ROWS_PER_W

  def fill(buf, n, value):
    def body(i, carry):
      buf[pl.ds(i * 16, 16)] = jnp.full((16,), value, jnp.int32)
      return carry
    lax.fori_loop(0, n // 16, body, 0)

  fill(zbuf, _SLICE, 0)
  fill(ones_buf, _PSTAGE, 1)
  fill(pst0, _PSTAGE, _VOCAB)
  fill(pst1, _PSTAGE, _VOCAB)

  pltpu.sync_copy(zbuf, pred_acc.at[pl.ds(s * _SLICE, _SLICE)])
  pltpu.sync_copy(zbuf, capt_acc.at[pl.ds(s * _SLICE, _SLICE)])
  plsc.subcore_barrier()

  sems = (sem0, sem1)

  def load_chunk(hbm, stage, sem, first_row, n_rows, width, stride):
    def body(r, carry):
      pltpu.async_copy(hbm.at[first_row + r, :],
                       stage.at[pl.ds(r * stride, width)], sem)
      return carry
    lax.fori_loop(0, n_rows, body, 0)

  def drain(stage, sem, n_words):
    pltpu.make_async_copy(pred_out.at[pl.ds(0, n_words)],
                          stage.at[pl.ds(0, n_words)], sem).wait()

  def scatter_input(hbm, acc, stages, n_rows_chunk, width, stride,
                    stage_words, load_words):
    n_chunks = _ROWS_PER_W // n_rows_chunk
    load_chunk(hbm, stages[0], sems[0], row0, n_rows_chunk, width, stride)
    for j in range(n_chunks):
      if j + 1 < n_chunks:
        load_chunk(hbm, stages[(j + 1) % 2], sems[(j + 1) % 2],
                   row0 + (j + 1) * n_rows_chunk, n_rows_chunk, width,
                   stride)
      drain(stages[j % 2], sems[j % 2], load_words)
      pltpu.sync_copy(ones_buf.at[pl.ds(0, stage_words)],
                      acc.at[stages[j % 2]], add=True)

  scatter_input(preds_hbm, pred_acc, (pst0, pst1), _PR, _PW, _PS,
                _PSTAGE, _PLOAD)
  scatter_input(capt_hbm, capt_acc, (cst0, cst1), _CR, _CW, _CW,
                _CSTAGE, _CSTAGE)
  plsc.subcore_barrier()

  off = c * _VP + s * _SLICE
  pltpu.sync_copy(pred_acc.at[pl.ds(s * _SLICE, _SLICE)],
                  pred_out.at[pl.ds(off, _SLICE)])
  pltpu.sync_copy(capt_acc.at[pl.ds(s * _SLICE, _SLICE)],
                  capt_out.at[pl.ds(off, _SLICE)])


_sc_hist = pl.kernel(
    _sc_hist_body,
    out_type=(
        jax.ShapeDtypeStruct((_NC * _VP,), jnp.int32),
        jax.ShapeDtypeStruct((_NC * _VP,), jnp.int32),
    ),
    mesh=plsc.VectorSubcoreMesh(core_axis_name="c", subcore_axis_name="s"),
    compiler_params=pltpu.CompilerParams(use_tc_tiling_on_sc=False),
    scratch_types=(
        pltpu.VMEM_SHARED((_VP,), jnp.int32),
        pltpu.VMEM_SHARED((_VP,), jnp.int32),
        pltpu.VMEM((_PSTAGE,), jnp.int32),
        pltpu.VMEM((_PSTAGE,), jnp.int32),
        pltpu.VMEM((_CSTAGE,), jnp.int32),
        pltpu.VMEM((_CSTAGE,), jnp.int32),
        pltpu.VMEM((_PSTAGE,), jnp.int32),
        pltpu.VMEM((_SLICE,), jnp.int32),
        pltpu.SemaphoreType.DMA,
        pltpu.SemaphoreType.DMA,
    ),
)


def _tc_merge_body(ph_ref, ch_ref, out_ref):
  valid = lax.broadcasted_iota(jnp.int32, (1, _VP), 1) < _VOCAB
  pm = (ph_ref[0:1, :] + ph_ref[1:2, :] > 0) & valid
  cm = (ch_ref[0:1, :] + ch_ref[1:2, :] > 0) & valid
  n_pred = jnp.sum(pm).astype(jnp.float32)
  n_capt = jnp.sum(cm).astype(jnp.float32)
  out_ref[0, 0] = jnp.where(
      n_capt > 0, n_pred / jnp.maximum(n_capt, 1.0), jnp.float32(0.0))


@jax.jit
def kernel(preds, captions):
  ph_flat, ch_flat = _sc_hist(preds, captions)
  ph = ph_flat.reshape(_NC, _VP)
  ch = ch_flat.reshape(_NC, _VP)
  ratio = pl.pallas_call(
      _tc_merge_body,
      out_shape=jax.ShapeDtypeStruct((1, 1), jnp.float32),
      in_specs=[
          pl.BlockSpec(memory_space=pltpu.VMEM),
          pl.BlockSpec(memory_space=pltpu.VMEM),
      ],
      out_specs=pl.BlockSpec(memory_space=pltpu.SMEM),
  )(ph, ch)
  return ratio[0, 0]


# speedup vs baseline: 1.2903x; 1.2903x over previous
import jax
import jax.numpy as jnp
from jax import lax
from jax.experimental import pallas as pl
from jax.experimental.pallas import tpu as pltpu
from jax.experimental.pallas import tpu_sc as plsc

_VOCAB = 100000
_NC = 2
_NS = 16
_NW = _NC * _NS
_VP = 100352
_SLICE = _VP // _NS

_N_PRED = 16384 * 50
_N_CAPT = 16384 * 200
_CHUNK = 12800
_PRED_PER_W = _N_PRED // _NW
_CAPT_PER_W = _N_CAPT // _NW


def _sc_hist_body(preds_hbm, capt_hbm, pred_out, capt_out,
                  pred_acc, capt_acc, idx0, idx1, ones_buf, zbuf,
                  sem0, sem1):
  c = lax.axis_index("c")
  s = lax.axis_index("s")
  w = c * _NS + s

  def fill_z(i, carry):
    zbuf[pl.ds(i * 16, 16)] = jnp.zeros((16,), jnp.int32)
    return carry

  def fill_o(i, carry):
    ones_buf[pl.ds(i * 16, 16)] = jnp.ones((16,), jnp.int32)
    return carry

  lax.fori_loop(0, _SLICE // 16, fill_z, 0)
  lax.fori_loop(0, _CHUNK // 16, fill_o, 0)

  pltpu.sync_copy(zbuf, pred_acc.at[pl.ds(s * _SLICE, _SLICE)])
  pltpu.sync_copy(zbuf, capt_acc.at[pl.ds(s * _SLICE, _SLICE)])
  plsc.subcore_barrier()

  bufs = (idx0, idx1)
  sems = (sem0, sem1)

  def scatter_input(hbm, acc, n_chunks, per_w):
    cps = [None] * n_chunks
    cps[0] = pltpu.async_copy(
        hbm.at[pl.ds(w * per_w, _CHUNK)], bufs[0], sems[0])
    for j in range(n_chunks):
      if j + 1 < n_chunks:
        base = w * per_w + (j + 1) * _CHUNK
        cps[j + 1] = pltpu.async_copy(
            hbm.at[pl.ds(base, _CHUNK)], bufs[(j + 1) % 2], sems[(j + 1) % 2])
      cps[j].wait()
      pltpu.sync_copy(ones_buf, acc.at[bufs[j % 2]], add=True)

  scatter_input(preds_hbm, pred_acc, _PRED_PER_W // _CHUNK, _PRED_PER_W)
  scatter_input(capt_hbm, capt_acc, _CAPT_PER_W // _CHUNK, _CAPT_PER_W)
  plsc.subcore_barrier()

  pltpu.sync_copy(pred_acc.at[pl.ds(s * _SLICE, _SLICE)],
                  pred_out.at[c, pl.ds(s * _SLICE, _SLICE)])
  pltpu.sync_copy(capt_acc.at[pl.ds(s * _SLICE, _SLICE)],
                  capt_out.at[c, pl.ds(s * _SLICE, _SLICE)])


_sc_hist = pl.kernel(
    _sc_hist_body,
    out_type=(
        jax.ShapeDtypeStruct((_NC, _VP), jnp.int32),
        jax.ShapeDtypeStruct((_NC, _VP), jnp.int32),
    ),
    mesh=plsc.VectorSubcoreMesh(core_axis_name="c", subcore_axis_name="s"),
    compiler_params=pltpu.CompilerParams(use_tc_tiling_on_sc=False),
    scratch_types=(
        pltpu.VMEM_SHARED((_VP,), jnp.int32),
        pltpu.VMEM_SHARED((_VP,), jnp.int32),
        pltpu.VMEM((_CHUNK,), jnp.int32),
        pltpu.VMEM((_CHUNK,), jnp.int32),
        pltpu.VMEM((_CHUNK,), jnp.int32),
        pltpu.VMEM((_SLICE,), jnp.int32),
        pltpu.SemaphoreType.DMA,
        pltpu.SemaphoreType.DMA,
    ),
)


def _tc_merge_body(ph_ref, ch_ref, out_ref):
  n_pred = jnp.sum((ph_ref[0] + ph_ref[1]) > 0).astype(jnp.float32)
  n_capt = jnp.sum((ch_ref[0] + ch_ref[1]) > 0).astype(jnp.float32)
  out_ref[0, 0] = jnp.where(
      n_capt > 0, n_pred / jnp.maximum(n_capt, 1.0), jnp.float32(0.0))


@jax.jit
def kernel(preds, captions):
  pf = preds.reshape(-1)
  cf = captions.reshape(-1)
  ph, ch = _sc_hist(pf, cf)
  ratio = pl.pallas_call(
      _tc_merge_body,
      out_shape=jax.ShapeDtypeStruct((1, 1), jnp.float32),
      in_specs=[
          pl.BlockSpec(memory_space=pltpu.VMEM),
          pl.BlockSpec(memory_space=pltpu.VMEM),
      ],
      out_specs=pl.BlockSpec(memory_space=pltpu.SMEM),
  )(ph, ch)
  return ratio[0, 0]


# speedup vs baseline: 1.3472x; 1.0440x over previous
import jax
import jax.numpy as jnp
from jax import lax
from jax.experimental import pallas as pl
from jax.experimental.pallas import tpu as pltpu
from jax.experimental.pallas import tpu_sc as plsc

_VOCAB = 100000
_NC = 2
_NS = 16
_NW = _NC * _NS
_VP = 100352
_SLICE = _VP // _NS

_N_PRED = 16384 * 50
_N_CAPT = 16384 * 200
_CHUNK = 12800
_PRED_PER_W = _N_PRED // _NW
_CAPT_PER_W = _N_CAPT // _NW


def _sc_hist_body(preds_hbm, capt_hbm, pred_out, capt_out,
                  pred_acc, capt_acc, idx0, idx1, ones_buf, zbuf,
                  sem0, sem1):
  c = lax.axis_index("c")
  s = lax.axis_index("s")
  w = c * _NS + s

  def fill_z(i, carry):
    zbuf[pl.ds(i * 16, 16)] = jnp.zeros((16,), jnp.int32)
    return carry

  def fill_o(i, carry):
    ones_buf[pl.ds(i * 16, 16)] = jnp.ones((16,), jnp.int32)
    return carry

  lax.fori_loop(0, _SLICE // 16, fill_z, 0)
  lax.fori_loop(0, _CHUNK // 16, fill_o, 0)

  pltpu.sync_copy(zbuf, pred_acc.at[pl.ds(s * _SLICE, _SLICE)])
  pltpu.sync_copy(zbuf, capt_acc.at[pl.ds(s * _SLICE, _SLICE)])
  plsc.subcore_barrier()

  bufs = (idx0, idx1)
  sems = (sem0, sem1)

  def scatter_input(hbm, acc, n_chunks, per_w):
    cps = [None] * n_chunks
    cps[0] = pltpu.async_copy(
        hbm.at[pl.ds(w * per_w, _CHUNK)], bufs[0], sems[0])
    for j in range(n_chunks):
      if j + 1 < n_chunks:
        base = w * per_w + (j + 1) * _CHUNK
        cps[j + 1] = pltpu.async_copy(
            hbm.at[pl.ds(base, _CHUNK)], bufs[(j + 1) % 2], sems[(j + 1) % 2])
      cps[j].wait()
      pltpu.sync_copy(ones_buf, acc.at[bufs[j % 2]], add=True)

  scatter_input(preds_hbm, pred_acc, _PRED_PER_W // _CHUNK, _PRED_PER_W)
  scatter_input(capt_hbm, capt_acc, _CAPT_PER_W // _CHUNK, _CAPT_PER_W)
  plsc.subcore_barrier()

  pltpu.sync_copy(pred_acc.at[pl.ds(s * _SLICE, _SLICE)],
                  pred_out.at[c, pl.ds(s * _SLICE, _SLICE)])
  pltpu.sync_copy(capt_acc.at[pl.ds(s * _SLICE, _SLICE)],
                  capt_out.at[c, pl.ds(s * _SLICE, _SLICE)])


_sc_hist = pl.kernel(
    _sc_hist_body,
    out_type=(
        jax.ShapeDtypeStruct((_NC, _VP), jnp.int32),
        jax.ShapeDtypeStruct((_NC, _VP), jnp.int32),
    ),
    mesh=plsc.VectorSubcoreMesh(core_axis_name="c", subcore_axis_name="s"),
    scratch_types=(
        pltpu.VMEM_SHARED((_VP,), jnp.int32),
        pltpu.VMEM_SHARED((_VP,), jnp.int32),
        pltpu.VMEM((_CHUNK,), jnp.int32),
        pltpu.VMEM((_CHUNK,), jnp.int32),
        pltpu.VMEM((_CHUNK,), jnp.int32),
        pltpu.VMEM((_SLICE,), jnp.int32),
        pltpu.SemaphoreType.DMA,
        pltpu.SemaphoreType.DMA,
    ),
)


def _tc_merge_body(ph_ref, ch_ref, out_ref):
  n_pred = jnp.sum((ph_ref[0] + ph_ref[1]) > 0).astype(jnp.float32)
  n_capt = jnp.sum((ch_ref[0] + ch_ref[1]) > 0).astype(jnp.float32)
  out_ref[0, 0] = jnp.where(
      n_capt > 0, n_pred / jnp.maximum(n_capt, 1.0), jnp.float32(0.0))


@jax.jit
def kernel(preds, captions):
  pf = preds.reshape(-1)
  cf = captions.reshape(-1)
  ph, ch = _sc_hist(pf, cf)
  ratio = pl.pallas_call(
      _tc_merge_body,
      out_shape=jax.ShapeDtypeStruct((1, 1), jnp.float32),
      in_specs=[
          pl.BlockSpec(memory_space=pltpu.VMEM),
          pl.BlockSpec(memory_space=pltpu.VMEM),
      ],
      out_specs=pl.BlockSpec(memory_space=pltpu.SMEM),
  )(ph, ch)
  return ratio[0, 0]


# speedup vs baseline: 1.4425x; 1.0708x over previous
import jax
import jax.numpy as jnp
from jax import lax
from jax.experimental import pallas as pl
from jax.experimental.pallas import tpu as pltpu
from jax.experimental.pallas import tpu_sc as plsc

_VOCAB = 100000
_NC = 2
_NS = 16
_NW = _NC * _NS
_VP = 100352
_SLICE = _VP // _NS

_N_PRED = 16384 * 50
_N_CAPT = 16384 * 200
_CHUNK = 12800
_PRED_PER_W = _N_PRED // _NW
_CAPT_PER_W = _N_CAPT // _NW


def _sc_hist_body(preds_hbm, capt_hbm, pred_out, capt_out,
                  pred_acc, capt_acc, idx0, idx1, ones_buf, zbuf,
                  sem0, sem1):
  c = lax.axis_index("c")
  s = lax.axis_index("s")
  w = c * _NS + s

  def fill_z(i, carry):
    zbuf[pl.ds(i * 16, 16)] = jnp.zeros((16,), jnp.int32)
    return carry

  def fill_o(i, carry):
    ones_buf[pl.ds(i * 16, 16)] = jnp.ones((16,), jnp.int32)
    return carry

  lax.fori_loop(0, _SLICE // 16, fill_z, 0)
  lax.fori_loop(0, _CHUNK // 16, fill_o, 0)

  pltpu.sync_copy(zbuf, pred_acc.at[pl.ds(s * _SLICE, _SLICE)])
  pltpu.sync_copy(zbuf, capt_acc.at[pl.ds(s * _SLICE, _SLICE)])
  plsc.subcore_barrier()

  bufs = (idx0, idx1)
  sems = (sem0, sem1)

  def scatter_input(hbm, acc, n_chunks):
    cps = [None] * n_chunks
    cps[0] = pltpu.async_copy(hbm.at[w * n_chunks, :], bufs[0], sems[0])
    for j in range(n_chunks):
      if j + 1 < n_chunks:
        cps[j + 1] = pltpu.async_copy(
            hbm.at[w * n_chunks + j + 1, :], bufs[(j + 1) % 2],
            sems[(j + 1) % 2])
      cps[j].wait()
      pltpu.sync_copy(ones_buf, acc.at[bufs[j % 2]], add=True)

  scatter_input(preds_hbm, pred_acc, _PRED_PER_W // _CHUNK)
  scatter_input(capt_hbm, capt_acc, _CAPT_PER_W // _CHUNK)
  plsc.subcore_barrier()

  pltpu.sync_copy(pred_acc.at[pl.ds(s * _SLICE, _SLICE)],
                  pred_out.at[c, pl.ds(s * _SLICE, _SLICE)])
  pltpu.sync_copy(capt_acc.at[pl.ds(s * _SLICE, _SLICE)],
                  capt_out.at[c, pl.ds(s * _SLICE, _SLICE)])


_sc_hist = pl.kernel(
    _sc_hist_body,
    out_type=(
        jax.ShapeDtypeStruct((_NC, _VP), jnp.int32),
        jax.ShapeDtypeStruct((_NC, _VP), jnp.int32),
    ),
    mesh=plsc.VectorSubcoreMesh(core_axis_name="c", subcore_axis_name="s"),
    scratch_types=(
        pltpu.VMEM_SHARED((_VP,), jnp.int32),
        pltpu.VMEM_SHARED((_VP,), jnp.int32),
        pltpu.VMEM((_CHUNK,), jnp.int32),
        pltpu.VMEM((_CHUNK,), jnp.int32),
        pltpu.VMEM((_CHUNK,), jnp.int32),
        pltpu.VMEM((_SLICE,), jnp.int32),
        pltpu.SemaphoreType.DMA,
        pltpu.SemaphoreType.DMA,
    ),
)


def _tc_merge_body(ph_ref, ch_ref, out_ref):
  n_pred = jnp.sum((ph_ref[0] + ph_ref[1]) > 0).astype(jnp.float32)
  n_capt = jnp.sum((ch_ref[0] + ch_ref[1]) > 0).astype(jnp.float32)
  out_ref[0, 0] = jnp.where(
      n_capt > 0, n_pred / jnp.maximum(n_capt, 1.0), jnp.float32(0.0))


@jax.jit
def kernel(preds, captions):
  pf = preds.reshape(-1, _CHUNK)
  cf = captions.reshape(-1, _CHUNK)
  ph, ch = _sc_hist(pf, cf)
  ratio = pl.pallas_call(
      _tc_merge_body,
      out_shape=jax.ShapeDtypeStruct((1, 1), jnp.float32),
      in_specs=[
          pl.BlockSpec(memory_space=pltpu.VMEM),
          pl.BlockSpec(memory_space=pltpu.VMEM),
      ],
      out_specs=pl.BlockSpec(memory_space=pltpu.SMEM),
  )(ph, ch)
  return ratio[0, 0]


# speedup vs baseline: 2.1502x; 1.4906x over previous
import jax
import jax.numpy as jnp
from jax import lax
from jax.experimental import pallas as pl
from jax.experimental.pallas import tpu as pltpu
from jax.experimental.pallas import tpu_sc as plsc

_VOCAB = 100000
_NC = 2
_NS = 16
_NW = _NC * _NS
_VP = 100352
_SLICE = _VP // _NS

_COLS = 16384
_P_ROWS = 50
_C_ROWS = 200
_P_CB = 1024
_C_CB = 4096
_P_BPR = _COLS // _P_CB
_C_BPR = _COLS // _C_CB
_TASKS_PER_W = 25


def _sc_hist_body(preds_hbm, capt_hbm, pred_out, capt_out,
                  pred_acc, capt_acc,
                  pst0, pst1, cst0, cst1, ones_buf, zbuf,
                  sem0, sem1):
  c = lax.axis_index("c")
  s = lax.axis_index("s")
  w = c * _NS + s
  t0 = w * _TASKS_PER_W

  def fill(buf, n, value):
    def body(i, carry):
      buf[pl.ds(i * 16, 16)] = jnp.full((16,), value, jnp.int32)
      return carry
    lax.fori_loop(0, n // 16, body, 0)

  fill(zbuf, _SLICE, 0)
  fill(ones_buf, _C_CB, 1)

  pltpu.sync_copy(zbuf, pred_acc.at[pl.ds(s * _SLICE, _SLICE)])
  pltpu.sync_copy(zbuf, capt_acc.at[pl.ds(s * _SLICE, _SLICE)])
  plsc.subcore_barrier()

  sems = (sem0, sem1)

  def scatter_input(hbm, acc, bufs, blocks_per_row, cb):
    def load(k, which):
      t = t0 + k
      row = t // blocks_per_row
      col = (t % blocks_per_row) * cb
      return pltpu.async_copy(hbm.at[row, pl.ds(col, cb)], bufs[which],
                              sems[which])

    cps = [None] * _TASKS_PER_W
    cps[0] = load(0, 0)
    for k in range(_TASKS_PER_W):
      if k + 1 < _TASKS_PER_W:
        cps[k + 1] = load(k + 1, (k + 1) % 2)
      cps[k].wait()
      pltpu.sync_copy(ones_buf.at[pl.ds(0, cb)], acc.at[bufs[k % 2]],
                      add=True)

  scatter_input(preds_hbm, pred_acc, (pst0, pst1), _P_BPR, _P_CB)
  scatter_input(capt_hbm, capt_acc, (cst0, cst1), _C_BPR, _C_CB)
  plsc.subcore_barrier()

  pltpu.sync_copy(pred_acc.at[pl.ds(s * _SLICE, _SLICE)],
                  pred_out.at[c, pl.ds(s * _SLICE, _SLICE)])
  pltpu.sync_copy(capt_acc.at[pl.ds(s * _SLICE, _SLICE)],
                  capt_out.at[c, pl.ds(s * _SLICE, _SLICE)])


_sc_hist = pl.kernel(
    _sc_hist_body,
    out_type=(
        jax.ShapeDtypeStruct((_NC, _VP), jnp.int32),
        jax.ShapeDtypeStruct((_NC, _VP), jnp.int32),
    ),
    mesh=plsc.VectorSubcoreMesh(core_axis_name="c", subcore_axis_name="s"),
    scratch_types=(
        pltpu.VMEM_SHARED((_VP,), jnp.int32),
        pltpu.VMEM_SHARED((_VP,), jnp.int32),
        pltpu.VMEM((_P_CB,), jnp.int32),
        pltpu.VMEM((_P_CB,), jnp.int32),
        pltpu.VMEM((_C_CB,), jnp.int32),
        pltpu.VMEM((_C_CB,), jnp.int32),
        pltpu.VMEM((_C_CB,), jnp.int32),
        pltpu.VMEM((_SLICE,), jnp.int32),
        pltpu.SemaphoreType.DMA,
        pltpu.SemaphoreType.DMA,
    ),
)


def _tc_merge_body(ph_ref, ch_ref, out_ref):
  n_pred = jnp.sum((ph_ref[0] + ph_ref[1]) > 0).astype(jnp.float32)
  n_capt = jnp.sum((ch_ref[0] + ch_ref[1]) > 0).astype(jnp.float32)
  out_ref[0, 0] = jnp.where(
      n_capt > 0, n_pred / jnp.maximum(n_capt, 1.0), jnp.float32(0.0))


@jax.jit
def kernel(preds, captions):
  ph, ch = _sc_hist(preds.T, captions.T)
  ratio = pl.pallas_call(
      _tc_merge_body,
      out_shape=jax.ShapeDtypeStruct((1, 1), jnp.float32),
      in_specs=[
          pl.BlockSpec(memory_space=pltpu.VMEM),
          pl.BlockSpec(memory_space=pltpu.VMEM),
      ],
      out_specs=pl.BlockSpec(memory_space=pltpu.SMEM),
  )(ph, ch)
  return ratio[0, 0]
